# Initial kernel scaffold; baseline (speedup 1.0000x reference)
import jax
import jax.numpy as jnp
from jax.experimental import pallas as pl


def kernel(H_source, A, W, b):
    raise NotImplementedError("write your pallas kernel here")


# speedup vs baseline: 1.8899x; 1.8899x over previous
import functools

import jax
import jax.numpy as jnp
from jax.experimental import pallas as pl


def _gcn_block(a_ref, h_ref, w_ref, b_ref, out_ref):
    a = a_ref[...]
    rs = jnp.sum(a, axis=1, keepdims=True)
    rs = jnp.maximum(rs, 1e-8)
    msg = jnp.dot(a, h_ref[...], preferred_element_type=jnp.float32)
    out = jnp.dot(msg / rs, w_ref[...].T, preferred_element_type=jnp.float32)
    out_ref[...] = out + b_ref[...]


@functools.partial(jax.jit, static_argnames=("bm",))
def _gcn(H_source, A, W, b2, bm):
    n_tgt, n_src = A.shape
    d_out = W.shape[0]
    return pl.pallas_call(
        _gcn_block,
        grid=(n_tgt // bm,),
        in_specs=[
            pl.BlockSpec((bm, n_src), lambda i: (i, 0)),
            pl.BlockSpec((n_src, H_source.shape[1]), lambda i: (0, 0)),
            pl.BlockSpec(W.shape, lambda i: (0, 0)),
            pl.BlockSpec(b2.shape, lambda i: (0, 0)),
        ],
        out_specs=pl.BlockSpec((bm, d_out), lambda i: (i, 0)),
        out_shape=jax.ShapeDtypeStruct((n_tgt, d_out), jnp.float32),
    )(A, H_source, W, b2)


def kernel(H_source, A, W, b):
    return _gcn(H_source, A, W, b.reshape(1, -1), bm=256)


# speedup vs baseline: 1.8915x; 1.0008x over previous
import functools

import jax
import jax.numpy as jnp
from jax.experimental import pallas as pl
from jax.experimental.pallas import tpu as pltpu


def _gcn_block(a_ref, h_ref, w_ref, b_ref, out_ref):
    a = a_ref[...]
    rs = jnp.sum(a, axis=1, keepdims=True)
    rs = jnp.maximum(rs, 1e-8)
    msg = jnp.dot(a, h_ref[...], preferred_element_type=jnp.float32)
    out = jnp.dot(msg / rs, w_ref[...].T, preferred_element_type=jnp.float32)
    out_ref[...] = out + b_ref[...]


@functools.partial(jax.jit, static_argnames=("bm",))
def _gcn(H_source, A, W, b2, bm):
    n_tgt, n_src = A.shape
    d_out = W.shape[0]
    return pl.pallas_call(
        _gcn_block,
        grid=(n_tgt // bm,),
        in_specs=[
            pl.BlockSpec((bm, n_src), lambda i: (i, 0)),
            pl.BlockSpec((n_src, H_source.shape[1]), lambda i: (0, 0)),
            pl.BlockSpec(W.shape, lambda i: (0, 0)),
            pl.BlockSpec(b2.shape, lambda i: (0, 0)),
        ],
        out_specs=pl.BlockSpec((bm, d_out), lambda i: (i, 0)),
        out_shape=jax.ShapeDtypeStruct((n_tgt, d_out), jnp.float32),
        compiler_params=pltpu.CompilerParams(
            dimension_semantics=("parallel",),
        ),
    )(A, H_source, W, b2)


def kernel(H_source, A, W, b):
    return _gcn(H_source, A, W, b.reshape(1, -1), bm=256)
